# baseline (speedup 1.0000x reference)
import functools

import jax
import jax.numpy as jnp
from jax.experimental import pallas as pl
from jax.experimental.pallas import tpu as pltpu

E = 8
TOPK = 2
D = 768
H = 4 * D
T = 2048
TM = 512


def _gelu_exact(h):
    return 0.5 * h * (1.0 + jax.lax.erf(h * 0.7071067811865476))


def _router_body(x_ref, gW_ref, gb_ref, bias_ref, gatew_ref):
    x = x_ref[...]
    logits = jnp.dot(x, gW_ref[...], preferred_element_type=jnp.float32)
    logits = logits + gb_ref[...] + bias_ref[...]
    s = jax.nn.sigmoid(logits)
    e_iota = jax.lax.broadcasted_iota(jnp.int32, s.shape, 1)
    m1 = jnp.max(s, axis=1, keepdims=True)
    i1 = jnp.min(jnp.where(s == m1, e_iota, E), axis=1, keepdims=True)
    s2 = jnp.where(e_iota == i1, -jnp.inf, s)
    m2 = jnp.max(s2, axis=1, keepdims=True)
    i2 = jnp.min(jnp.where(s2 == m2, e_iota, E), axis=1, keepdims=True)
    mask = (e_iota == i1) | (e_iota == i2)
    gatew_ref[...] = jnp.where(mask, s, 0.0)


def _moe_dense_body(gatew_ref, x_ref, eW1_ref, eb1_ref, eW2_ref, eb2_ref,
                    sW1_ref, sb1_ref, sW2_ref, sb2_ref, out_ref):
    e = pl.program_id(0)
    hh = pl.program_id(1)
    t = pl.program_id(2)
    is_shared = e == E

    xb = x_ref[...]
    w1 = jnp.where(is_shared, sW1_ref[...], eW1_ref[0])
    w2 = jnp.where(is_shared, sW2_ref[...], eW2_ref[0])
    b1 = jnp.where(is_shared, sb1_ref[...], eb1_ref[0])
    b2 = jnp.where(is_shared, sb2_ref[...], eb2_ref[0])
    h = jnp.dot(xb, w1, preferred_element_type=jnp.float32) + b1
    h = _gelu_exact(h)
    y = jnp.dot(h, w2, preferred_element_type=jnp.float32)
    y = y + jnp.where(hh == 0, b2, 0.0)

    gw = gatew_ref[...]
    e_iota = jax.lax.broadcasted_iota(jnp.int32, gw.shape, 1)
    scale = jnp.sum(jnp.where(e_iota == e, gw, 0.0), axis=1, keepdims=True)
    scale = jnp.where(is_shared, 1.0, scale)
    contrib = y * scale

    sl = pl.ds(t * TM, TM)
    first = (e == 0) & (hh == 0)

    @pl.when(first)
    def _():
        out_ref[sl, :] = contrib

    @pl.when(jnp.logical_not(first))
    def _():
        out_ref[sl, :] = out_ref[sl, :] + contrib


def _router(x_flat, gate_W, gate_b, bias, interpret=False):
    return pl.pallas_call(
        _router_body,
        out_shape=jax.ShapeDtypeStruct((T, E), jnp.float32),
        interpret=interpret,
    )(x_flat, gate_W, gate_b.reshape(1, E), bias.reshape(1, E))


def _moe_dense(gate_w, x_flat, eW1, eb1, eW2, eb2, sW1, sb1, sW2, sb2,
               interpret=False):
    H2 = H // 2
    grid = (E + 1, 2, T // TM)
    ec = lambda e: jnp.minimum(e, E - 1)
    return pl.pallas_call(
        _moe_dense_body,
        grid=grid,
        in_specs=[
            pl.BlockSpec((TM, E), lambda e, hh, t: (t, 0)),
            pl.BlockSpec((TM, D), lambda e, hh, t: (t, 0)),
            pl.BlockSpec((1, D, H2), lambda e, hh, t: (ec(e), 0, hh)),
            pl.BlockSpec((1, 1, H2), lambda e, hh, t: (ec(e), 0, hh)),
            pl.BlockSpec((1, H2, D), lambda e, hh, t: (ec(e), hh, 0)),
            pl.BlockSpec((1, 1, D), lambda e, hh, t: (ec(e), 0, 0)),
            pl.BlockSpec((D, H2), lambda e, hh, t: (0, hh)),
            pl.BlockSpec((1, H2), lambda e, hh, t: (0, hh)),
            pl.BlockSpec((H2, D), lambda e, hh, t: (hh, 0)),
            pl.BlockSpec((1, D), lambda e, hh, t: (0, 0)),
        ],
        out_specs=pl.BlockSpec((T, D), lambda e, hh, t: (0, 0)),
        out_shape=jax.ShapeDtypeStruct((T, D), jnp.float32),
        compiler_params=pltpu.CompilerParams(
            dimension_semantics=("arbitrary", "arbitrary", "arbitrary"),
        ),
        interpret=interpret,
    )(gate_w, x_flat, eW1, eb1, eW2, eb2, sW1, sb1, sW2, sb2)


def _kernel_impl(x, gate_W, gate_b, bias, sW1, sb1, sW2, sb2,
                 eW1, eb1, eW2, eb2, interpret=False):
    B, S, Dh = x.shape
    x_flat = x.reshape(-1, Dh)
    gate_w = _router(x_flat, gate_W, gate_b, bias, interpret=interpret)
    bf = jnp.bfloat16
    out = _moe_dense(
        gate_w, x_flat,
        eW1, eb1.reshape(E, 1, H), eW2,
        eb2.reshape(E, 1, D),
        sW1, sb1.reshape(1, H), sW2, sb2.reshape(1, D),
        interpret=interpret)
    return out.reshape(B, S, Dh)


def kernel(x, gate_W, gate_b, bias, sW1, sb1, sW2, sb2, eW1, eb1, eW2, eb2):
    return _kernel_impl(x, gate_W, gate_b, bias, sW1, sb1, sW2, sb2,
                        eW1, eb1, eW2, eb2)


# speedup vs baseline: 1.4335x; 1.4335x over previous
import functools

import jax
import jax.numpy as jnp
from jax import lax
from jax.experimental import pallas as pl
from jax.experimental.pallas import tpu as pltpu
from jax.experimental.pallas import tpu_sc as plsc

E = 8
TOPK = 2
D = 768
H = 4 * D
T = 2048
TME = 256
NT = T * TOPK // TME + E - 1
NPAD = NT * TME
NSH = T // TME
NW = 32
TPW = T // NW
HALF = 32


def _gelu_exact(h):
    return 0.5 * h * (1.0 + jax.lax.erf(h * 0.7071067811865476))


def _router_body(x_ref, gW_ref, gb_ref, pos0_ref, pos1_ref, w0_ref, w1_ref,
                 eot_ref, nt_ref):
    x = x_ref[...]
    logits = jnp.dot(x, gW_ref[...], preferred_element_type=jnp.float32)
    s = jax.nn.sigmoid(logits + gb_ref[...])
    e_iota = lax.broadcasted_iota(jnp.int32, s.shape, 1)
    m1 = jnp.max(s, axis=1, keepdims=True)
    i1 = jnp.min(jnp.where(s == m1, e_iota, E), axis=1, keepdims=True)
    s2 = jnp.where(e_iota == i1, -jnp.inf, s)
    m2 = jnp.max(s2, axis=1, keepdims=True)
    i2 = jnp.min(jnp.where(s2 == m2, e_iota, E), axis=1, keepdims=True)
    oh1 = e_iota == i1
    oh2 = e_iota == i2
    mask_f = (oh1 | oh2).astype(jnp.float32)

    r_iota = lax.broadcasted_iota(jnp.int32, (T, T), 0)
    c_iota = lax.broadcasted_iota(jnp.int32, (T, T), 1)
    L = (r_iota > c_iota).astype(jnp.float32)
    ranks = lax.dot_general(L, mask_f, (((1,), (0,)), ((), ())),
                            precision=lax.Precision.HIGHEST,
                            preferred_element_type=jnp.float32)
    counts = jnp.sum(mask_f, axis=0, keepdims=True)
    cnt_pad = jnp.ceil(counts * (1.0 / TME)) * TME
    u_r = lax.broadcasted_iota(jnp.int32, (E, E), 0)
    u_c = lax.broadcasted_iota(jnp.int32, (E, E), 1)
    U = (u_r < u_c).astype(jnp.float32)
    offsets = lax.dot_general(cnt_pad, U, (((1,), (0,)), ((), ())),
                              precision=lax.Precision.HIGHEST,
                              preferred_element_type=jnp.float32)

    posf = offsets + ranks
    pos0_ref[...] = jnp.sum(jnp.where(oh1, posf, 0.0), axis=1,
                            keepdims=True).astype(jnp.int32)
    pos1_ref[...] = jnp.sum(jnp.where(oh2, posf, 0.0), axis=1,
                            keepdims=True).astype(jnp.int32)
    w0_ref[...] = m1
    w1_ref[...] = m2

    ts = offsets * (1.0 / TME)
    j_iota = lax.broadcasted_iota(jnp.int32, (NW, E), 0).astype(jnp.float32)
    ge = (j_iota >= ts).astype(jnp.float32)
    eot_ref[...] = (jnp.sum(ge, axis=1, keepdims=True) - 1.0).astype(jnp.int32)
    nt_ref[...] = (jnp.sum(cnt_pad, axis=1, keepdims=True)
                   * (1.0 / TME)).astype(jnp.int32)


def _router(x_flat, gate_W, gbias, interpret=False):
    outs = pl.pallas_call(
        _router_body,
        out_shape=(
            jax.ShapeDtypeStruct((T, 1), jnp.int32),
            jax.ShapeDtypeStruct((T, 1), jnp.int32),
            jax.ShapeDtypeStruct((T, 1), jnp.float32),
            jax.ShapeDtypeStruct((T, 1), jnp.float32),
            jax.ShapeDtypeStruct((NW, 1), jnp.int32),
            jax.ShapeDtypeStruct((1, 1), jnp.int32),
        ),
        interpret=interpret,
    )(x_flat, gate_W, gbias)
    return outs


def _sc_dispatch_call(x_flat, p0, p1, w0, w1):
    @functools.partial(
        pl.kernel,
        out_type=(
            jax.ShapeDtypeStruct((NPAD, D), jnp.float32),
                jax.ShapeDtypeStruct((NPAD, 128), jnp.float32),
        ),
        mesh=plsc.VectorSubcoreMesh(core_axis_name="c", subcore_axis_name="s"),
        compiler_params=pltpu.CompilerParams(needs_layout_passes=False),
        scratch_types=[
            pltpu.VMEM((TPW, D), jnp.float32),
            pltpu.VMEM((TPW, 128), jnp.float32),
            pltpu.VMEM((TPW, 128), jnp.float32),
            pltpu.VMEM((TPW,), jnp.int32),
            pltpu.VMEM((TPW,), jnp.int32),
            pltpu.VMEM((TPW,), jnp.float32),
            pltpu.VMEM((TPW,), jnp.float32),
            pltpu.SemaphoreType.DMA,
            pltpu.SemaphoreType.DMA,
            pltpu.SemaphoreType.DMA,
            pltpu.SemaphoreType.DMA,
        ],
    )
    def _sc_dispatch(x_hbm, pos0_hbm, pos1_hbm, w0_hbm, w1_hbm, xs_hbm, ws_hbm,
                     xbuf, wbuf0, wbuf1, idx0, idx1, wv0, wv1,
                     sem0, sem1, sem2, sem3):
        wid = lax.axis_index("s") * 2 + lax.axis_index("c")
        base = wid * TPW
        pltpu.sync_copy(pos0_hbm.at[pl.ds(base, TPW)], idx0)
        pltpu.sync_copy(pos1_hbm.at[pl.ds(base, TPW)], idx1)
        pltpu.sync_copy(w0_hbm.at[pl.ds(base, TPW)], wv0)
        pltpu.sync_copy(w1_hbm.at[pl.ds(base, TPW)], wv1)
        pltpu.sync_copy(x_hbm.at[pl.ds(base, TPW)], xbuf)
        for r in range(TPW):
            ridx = jnp.full((16,), r, jnp.int32)
            wbuf0[r, pl.ds(0, 16)] = plsc.load_gather(wv0, [ridx])
            wbuf1[r, pl.ds(0, 16)] = plsc.load_gather(wv1, [ridx])
        c0 = pltpu.async_copy(xbuf, xs_hbm.at[idx0], sem0)
        c1 = pltpu.async_copy(xbuf, xs_hbm.at[idx1], sem1)
        c2 = pltpu.async_copy(wbuf0, ws_hbm.at[idx0], sem2)
        c3 = pltpu.async_copy(wbuf1, ws_hbm.at[idx1], sem3)
        c0.wait()
        c1.wait()
        c2.wait()
        c3.wait()

    return _sc_dispatch(x_flat, p0, p1, w0, w1)


def _mlp_body(eot_s, nt_s, xs_ref, x_ref, ws_ref,
              eW1_ref, eb1_ref, eW2_ref, eb2_ref,
              sW1_ref, sb1_ref, sW2_ref, sb2_ref, ys_ref):
    j = pl.program_id(0)

    @pl.when(j < nt_s[0])
    def _():
        xb = xs_ref[...]
        h = jnp.dot(xb, eW1_ref[0],
                    preferred_element_type=jnp.float32) + eb1_ref[0]
        h = _gelu_exact(h)
        y = jnp.dot(h, eW2_ref[0],
                    preferred_element_type=jnp.float32) + eb2_ref[0]
        ys_ref[...] = y * ws_ref[...][:, :1]

    @pl.when(j >= NT)
    def _():
        xb = x_ref[...].astype(jnp.bfloat16)
        h = jnp.dot(xb, sW1_ref[...],
                    preferred_element_type=jnp.float32) + sb1_ref[...]
        h = _gelu_exact(h).astype(jnp.bfloat16)
        y = jnp.dot(h, sW2_ref[...],
                    preferred_element_type=jnp.float32) + sb2_ref[...]
        ys_ref[...] = y


def _mlp(eot, nt, xs, x_flat, ws, eW1, eb1, eW2, eb2, sW1b, sb1, sW2b, sb2,
         interpret=False):
    ecl = lambda j: jnp.minimum(j, NT - 1)
    grid_spec = pltpu.PrefetchScalarGridSpec(
        num_scalar_prefetch=2,
        grid=(NT + NSH,),
        in_specs=[
            pl.BlockSpec((TME, D), lambda j, eot, nt: (ecl(j), 0)),
            pl.BlockSpec((TME, D), lambda j, eot, nt: (jnp.clip(j - NT, 0, NSH - 1), 0)),
            pl.BlockSpec((TME, 128), lambda j, eot, nt: (ecl(j), 0)),
            pl.BlockSpec((1, D, H), lambda j, eot, nt: (eot[ecl(j)], 0, 0)),
            pl.BlockSpec((1, 1, H), lambda j, eot, nt: (eot[ecl(j)], 0, 0)),
            pl.BlockSpec((1, H, D), lambda j, eot, nt: (eot[ecl(j)], 0, 0)),
            pl.BlockSpec((1, 1, D), lambda j, eot, nt: (eot[ecl(j)], 0, 0)),
            pl.BlockSpec((D, H), lambda j, eot, nt: (0, 0)),
            pl.BlockSpec((1, H), lambda j, eot, nt: (0, 0)),
            pl.BlockSpec((H, D), lambda j, eot, nt: (0, 0)),
            pl.BlockSpec((1, D), lambda j, eot, nt: (0, 0)),
        ],
        out_specs=pl.BlockSpec((TME, D), lambda j, eot, nt: (j, 0)),
    )
    return pl.pallas_call(
        _mlp_body,
        grid_spec=grid_spec,
        out_shape=jax.ShapeDtypeStruct((NPAD + T, D), jnp.float32),
        compiler_params=pltpu.CompilerParams(
            dimension_semantics=("arbitrary",),
            vmem_limit_bytes=62 * 1024 * 1024,
        ),
        interpret=interpret,
    )(eot, nt, xs, x_flat, ws, eW1, eb1, eW2, eb2, sW1b, sb1, sW2b, sb2)


def _sc_combine_call(ys, p0, p1):
    @functools.partial(
        pl.kernel,
        out_type=jax.ShapeDtypeStruct((T, D), jnp.float32),
        mesh=plsc.VectorSubcoreMesh(core_axis_name="c", subcore_axis_name="s"),
        compiler_params=pltpu.CompilerParams(needs_layout_passes=False),
        scratch_types=[
            pltpu.VMEM((HALF, D), jnp.float32),
            pltpu.VMEM((HALF, D), jnp.float32),
            pltpu.VMEM((HALF, D), jnp.float32),
            pltpu.VMEM((HALF,), jnp.int32),
            pltpu.VMEM((HALF,), jnp.int32),
            pltpu.SemaphoreType.DMA,
            pltpu.SemaphoreType.DMA,
            pltpu.SemaphoreType.DMA,
        ],
    )
    def _sc_combine(ys_hbm, pos0_hbm, pos1_hbm, out_hbm,
                    b0, b1, bs, i0, i1, s0, s1, s2):
        wid = lax.axis_index("s") * 2 + lax.axis_index("c")
        base = wid * TPW
        for c in range(TPW // HALF):
            off = base + c * HALF
            pltpu.sync_copy(pos0_hbm.at[pl.ds(off, HALF)], i0)
            pltpu.sync_copy(pos1_hbm.at[pl.ds(off, HALF)], i1)
            g0 = pltpu.async_copy(ys_hbm.at[i0], b0, s0)
            g1 = pltpu.async_copy(ys_hbm.at[i1], b1, s1)
            gs = pltpu.async_copy(ys_hbm.at[pl.ds(NPAD + off, HALF)], bs, s2)
            g0.wait()
            g1.wait()
            gs.wait()
            for r in range(HALF):
                def add_row(k, _, r=r):
                    sl = pl.ds(k * 16, 16)
                    b0[r, sl] = b0[r, sl] + b1[r, sl] + bs[r, sl]
                    return 0
                lax.fori_loop(0, D // 16, add_row, 0)
            pltpu.sync_copy(b0, out_hbm.at[pl.ds(off, HALF)])

    return _sc_combine(ys, p0, p1)


def _kernel_impl(x, gate_W, gate_b, bias, sW1, sb1, sW2, sb2,
                 eW1, eb1, eW2, eb2, interpret=False):
    B, S, Dh = x.shape
    x_flat = x.reshape(-1, Dh)
    gbias = (gate_b + bias).reshape(1, E)
    pos0, pos1, w0, w1, eot, nt = _router(x_flat, gate_W, gbias,
                                          interpret=interpret)
    p0 = pos0.reshape(T)
    p1 = pos1.reshape(T)
    xs, ws = _sc_dispatch_call(x_flat, p0, p1, w0.reshape(T), w1.reshape(T))
    bf = jnp.bfloat16
    ys = _mlp(eot.reshape(NW), nt.reshape(1), xs, x_flat, ws,
              eW1, eb1.reshape(E, 1, H), eW2, eb2.reshape(E, 1, D),
              sW1.astype(bf), sb1.reshape(1, H), sW2.astype(bf),
              sb2.reshape(1, D), interpret=interpret)
    out = _sc_combine_call(ys, p0, p1)
    return out.reshape(B, S, Dh)


def kernel(x, gate_W, gate_b, bias, sW1, sb1, sW2, sb2, eW1, eb1, eW2, eb2):
    return _kernel_impl(x, gate_W, gate_b, bias, sW1, sb1, sW2, sb2,
                        eW1, eb1, eW2, eb2)
